# Initial kernel scaffold; baseline (speedup 1.0000x reference)
import jax
import jax.numpy as jnp
from jax.experimental import pallas as pl


def kernel(hidden_states, gate_w, expert_gate, expert_up, expert_down, shared_gate_w, shared_up_w, shared_down_w):
    raise NotImplementedError("write your pallas kernel here")


# speedup vs baseline: 1.0514x; 1.0514x over previous
import jax
import jax.numpy as jnp
from jax.experimental import pallas as pl
from jax.experimental.pallas import tpu as pltpu

_BT = 256


def _router_body(x_ref, gate_w_ref, logits_ref, combine_ref):
    T = x_ref.shape[0]
    E = gate_w_ref.shape[0]
    logits = jax.lax.dot_general(
        x_ref[...], gate_w_ref[...], (((1,), (1,)), ((), ())),
        preferred_element_type=jnp.float32)
    logits_ref[...] = logits
    s = jax.nn.sigmoid(logits)
    eidx = jax.lax.broadcasted_iota(jnp.int32, (T, E), 1)
    s_left = jnp.roll(s, -1, axis=1)
    s_right = jnp.roll(s, 1, axis=1)
    even = (eidx % 2) == 0
    gsc = s + jnp.where(even, s_left, s_right)
    glane = eidx >> 1
    m1 = jnp.max(gsc, axis=1, keepdims=True)
    i1g = jnp.min(jnp.where(gsc == m1, glane, 99), axis=1, keepdims=True)
    g2 = jnp.where(glane == i1g, -jnp.inf, gsc)
    m2 = jnp.max(g2, axis=1, keepdims=True)
    i2g = jnp.min(jnp.where(g2 == m2, glane, 99), axis=1, keepdims=True)
    emask = (glane == i1g) | (glane == i2g)
    ms = s * emask.astype(jnp.float32)
    v1 = jnp.max(ms, axis=1, keepdims=True)
    j1 = jnp.min(jnp.where(ms == v1, eidx, 99), axis=1, keepdims=True)
    ms2 = jnp.where(eidx == j1, -jnp.inf, ms)
    v2 = jnp.max(ms2, axis=1, keepdims=True)
    j2 = jnp.min(jnp.where(ms2 == v2, eidx, 99), axis=1, keepdims=True)
    denom = jnp.maximum(v1 + v2, 1e-12)
    combine_ref[...] = (jnp.where(eidx == j1, v1 / denom, 0.0) +
                        jnp.where(eidx == j2, v2 / denom, 0.0))


def _moe_body(x_ref, combine_ref, wg_ref, wu_ref, wd_ref, out_ref):
    e = pl.program_id(1)
    x = x_ref[...]
    gate = jax.lax.dot_general(x, wg_ref[0], (((1,), (1,)), ((), ())),
                               preferred_element_type=jnp.float32)
    up = jax.lax.dot_general(x, wu_ref[0], (((1,), (1,)), ((), ())),
                             preferred_element_type=jnp.float32)
    h = gate * jax.nn.sigmoid(gate) * up
    y = jax.lax.dot_general(h, wd_ref[0], (((1,), (1,)), ((), ())),
                            preferred_element_type=jnp.float32)
    eidx = jax.lax.broadcasted_iota(jnp.int32, combine_ref.shape, 1)
    w = jnp.sum(jnp.where(eidx == e, combine_ref[...], 0.0), axis=1,
                keepdims=True)
    acc = y * w

    @pl.when(e == 0)
    def _init():
        out_ref[...] = acc

    @pl.when(e != 0)
    def _accum():
        out_ref[...] += acc


def _shared_body(x_ref, moe_ref, wg_ref, wu_ref, wd_ref, out_ref):
    x = x_ref[...]
    gate = jax.lax.dot_general(x, wg_ref[...], (((1,), (1,)), ((), ())),
                               preferred_element_type=jnp.float32)
    up = jax.lax.dot_general(x, wu_ref[...], (((1,), (1,)), ((), ())),
                             preferred_element_type=jnp.float32)
    h = gate * jax.nn.sigmoid(gate) * up
    y = jax.lax.dot_general(h, wd_ref[...], (((1,), (1,)), ((), ())),
                            preferred_element_type=jnp.float32)
    out_ref[...] = y + moe_ref[...]


def kernel(hidden_states, gate_w, expert_gate, expert_up, expert_down,
           shared_gate_w, shared_up_w, shared_down_w):
    bsz, seq, dim = hidden_states.shape
    T = bsz * seq
    E, I, D = expert_gate.shape
    SH = shared_gate_w.shape[0]
    flat = hidden_states.reshape(T, D)
    BT = _BT

    logits, combine = pl.pallas_call(
        _router_body,
        in_specs=[pl.BlockSpec((T, D), lambda: (0, 0)),
                  pl.BlockSpec((E, D), lambda: (0, 0))],
        out_specs=[pl.BlockSpec((T, E), lambda: (0, 0)),
                   pl.BlockSpec((T, E), lambda: (0, 0))],
        out_shape=[jax.ShapeDtypeStruct((T, E), jnp.float32),
                   jax.ShapeDtypeStruct((T, E), jnp.float32)],
    )(flat, gate_w)

    moe_out = pl.pallas_call(
        _moe_body,
        grid=(T // BT, E),
        in_specs=[
            pl.BlockSpec((BT, D), lambda j, e: (j, 0)),
            pl.BlockSpec((BT, E), lambda j, e: (j, 0)),
            pl.BlockSpec((1, I, D), lambda j, e: (e, 0, 0)),
            pl.BlockSpec((1, I, D), lambda j, e: (e, 0, 0)),
            pl.BlockSpec((1, D, I), lambda j, e: (e, 0, 0)),
        ],
        out_specs=pl.BlockSpec((BT, D), lambda j, e: (j, 0)),
        out_shape=jax.ShapeDtypeStruct((T, D), jnp.float32),
        compiler_params=pltpu.CompilerParams(
            dimension_semantics=("parallel", "arbitrary")),
    )(flat, combine, expert_gate, expert_up, expert_down)

    out = pl.pallas_call(
        _shared_body,
        grid=(T // BT,),
        in_specs=[
            pl.BlockSpec((BT, D), lambda j: (j, 0)),
            pl.BlockSpec((BT, D), lambda j: (j, 0)),
            pl.BlockSpec((SH, D), lambda j: (0, 0)),
            pl.BlockSpec((SH, D), lambda j: (0, 0)),
            pl.BlockSpec((D, SH), lambda j: (0, 0)),
        ],
        out_specs=pl.BlockSpec((BT, D), lambda j: (j, 0)),
        out_shape=jax.ShapeDtypeStruct((T, D), jnp.float32),
        compiler_params=pltpu.CompilerParams(
            dimension_semantics=("parallel",)),
    )(flat, moe_out, shared_gate_w, shared_up_w, shared_down_w)

    return out.reshape(bsz, seq, dim), logits


# speedup vs baseline: 1.8433x; 1.7532x over previous
import functools

import jax
import jax.numpy as jnp
from jax import lax
from jax.experimental import pallas as pl
from jax.experimental.pallas import tpu as pltpu
from jax.experimental.pallas import tpu_sc as plsc

_BT = 256
_BTS = 256


def _router_body(x_ref, gate_w_ref, logits_ref, pos_ref, wn_ref, te_ref,
                 act_ref):
    T = x_ref.shape[0]
    E = gate_w_ref.shape[0]
    BTS = _BTS
    NT = te_ref.shape[0]
    logits = jax.lax.dot_general(
        x_ref[...], gate_w_ref[...], (((1,), (1,)), ((), ())),
        preferred_element_type=jnp.float32)
    logits_ref[...] = logits
    s = jax.nn.sigmoid(logits)
    eidx = jax.lax.broadcasted_iota(jnp.int32, (T, E), 1)
    s_left = jnp.roll(s, -1, axis=1)
    s_right = jnp.roll(s, 1, axis=1)
    even = (eidx % 2) == 0
    gsc = s + jnp.where(even, s_left, s_right)
    glane = eidx >> 1
    m1 = jnp.max(gsc, axis=1, keepdims=True)
    i1g = jnp.min(jnp.where(gsc == m1, glane, 99), axis=1, keepdims=True)
    g2 = jnp.where(glane == i1g, -jnp.inf, gsc)
    m2 = jnp.max(g2, axis=1, keepdims=True)
    i2g = jnp.min(jnp.where(g2 == m2, glane, 99), axis=1, keepdims=True)
    emask = (glane == i1g) | (glane == i2g)
    ms = s * emask.astype(jnp.float32)
    v1 = jnp.max(ms, axis=1, keepdims=True)
    j1 = jnp.min(jnp.where(ms == v1, eidx, 99), axis=1, keepdims=True)
    ms2 = jnp.where(eidx == j1, -jnp.inf, ms)
    v2 = jnp.max(ms2, axis=1, keepdims=True)
    j2 = jnp.min(jnp.where(ms2 == v2, eidx, 99), axis=1, keepdims=True)
    denom = jnp.maximum(v1 + v2, 1e-12)
    w1n = v1 / denom
    w2n = v2 / denom
    wn_ref[...] = jnp.concatenate([w1n, w2n], axis=1)

    H1 = (eidx == j1).astype(jnp.int32)
    H2 = (eidx == j2).astype(jnp.int32)
    Hsum = (H1 + H2).astype(jnp.float32)
    CB = 256
    ri = jax.lax.broadcasted_iota(jnp.int32, (CB, CB), 0)
    ci = jax.lax.broadcasted_iota(jnp.int32, (CB, CB), 1)
    tril = (ri > ci).astype(jnp.bfloat16)
    chunks = []
    carry = jnp.zeros((1, E), jnp.float32)
    for c in range(T // CB):
        Hc = Hsum[c * CB:(c + 1) * CB, :]
        Cc = jax.lax.dot_general(
            tril, Hc.astype(jnp.bfloat16), (((1,), (0,)), ((), ())),
            preferred_element_type=jnp.float32)
        chunks.append(Cc + carry)
        carry = carry + jnp.sum(Hc, axis=0, keepdims=True)
    C = jnp.concatenate(chunks, axis=0).astype(jnp.int32)
    cnt = carry.astype(jnp.int32)
    pc = ((cnt + (BTS - 1)) // BTS) * BTS
    lane8 = jax.lax.broadcasted_iota(jnp.int32, (1, E), 1)
    incl = pc
    for k in (1, 2, 4):
        sh = jnp.where(lane8 >= k, jnp.roll(incl, k, axis=1), 0)
        incl = incl + sh
    off = incl - pc
    rank1 = jnp.sum(H1 * C, axis=1, keepdims=True)
    rank2 = jnp.sum(H2 * C, axis=1, keepdims=True)
    pos1 = jnp.sum(H1 * off, axis=1, keepdims=True) + rank1
    pos2 = jnp.sum(H2 * off, axis=1, keepdims=True) + rank2
    pos_ref[...] = jnp.concatenate([pos1, pos2], axis=1)

    jrow = jax.lax.broadcasted_iota(jnp.int32, (NT, E), 0) * BTS
    offb = jnp.broadcast_to(off, (NT, E))
    pcb = jnp.broadcast_to(pc, (NT, E))
    eb = jax.lax.broadcasted_iota(jnp.int32, (NT, E), 1)
    inseg = (jrow >= offb) & (jrow < offb + pcb)
    te_col = jnp.sum(jnp.where(inseg, eb, 0), axis=1, keepdims=True)
    total = jnp.sum(pc, axis=1, keepdims=True)
    te_last = jnp.max(jnp.where(pc > 0, lane8, 0), axis=1, keepdims=True)
    act_col = (jrow[:, 0:1] < jnp.broadcast_to(total, (NT, 1)))
    te_ref[...] = jnp.where(act_col, te_col, jnp.broadcast_to(te_last, (NT, 1)))
    act_ref[...] = act_col.astype(jnp.int32)


def _ffn_body(te_ref, act_ref, xs_ref, wg_ref, wu_ref, wd_ref,
              out_ref):
    j = pl.program_id(0)

    @pl.when(act_ref[j] == 1)
    def _():
        x = xs_ref[...]
        gate = jax.lax.dot_general(x, wg_ref[0], (((1,), (1,)), ((), ())),
                                   preferred_element_type=jnp.float32)
        up = jax.lax.dot_general(x, wu_ref[0], (((1,), (1,)), ((), ())),
                                 preferred_element_type=jnp.float32)
        h = gate * jax.nn.sigmoid(gate) * up
        y = jax.lax.dot_general(h, wd_ref[0], (((1,), (1,)), ((), ())),
                                preferred_element_type=jnp.float32)
        out_ref[...] = y


def _shared_body(x_ref, moe1_ref, moe2_ref, w1_ref, w2_ref, wg_ref, wu_ref,
                 wd_ref, out_ref):
    x = x_ref[...]
    gate = jax.lax.dot_general(x, wg_ref[...], (((1,), (1,)), ((), ())),
                               preferred_element_type=jnp.float32)
    up = jax.lax.dot_general(x, wu_ref[...], (((1,), (1,)), ((), ())),
                             preferred_element_type=jnp.float32)
    h = gate * jax.nn.sigmoid(gate) * up
    y = jax.lax.dot_general(h, wd_ref[...], (((1,), (1,)), ((), ())),
                            preferred_element_type=jnp.float32)
    out_ref[...] = (moe1_ref[...] * w1_ref[...] +
                    moe2_ref[...] * w2_ref[...]) + y


def kernel(hidden_states, gate_w, expert_gate, expert_up, expert_down,
           shared_gate_w, shared_up_w, shared_down_w):
    bsz, seq, dim = hidden_states.shape
    T = bsz * seq
    E, I, D = expert_gate.shape
    SH = shared_gate_w.shape[0]
    flat = hidden_states.reshape(T, D)
    BT, BTS = _BT, _BTS
    TK = 2 * T
    NT = TK // BTS + E
    P = NT * BTS

    NC, NS = 2, 16
    NW = NC * NS
    mesh = plsc.VectorSubcoreMesh(core_axis_name="c", subcore_axis_name="s",
                                  num_cores=NC, num_subcores=NS)

    logits, pos2d, wn2d, te2d, act2d = pl.pallas_call(
        _router_body,
        out_shape=[
            jax.ShapeDtypeStruct((T, E), jnp.float32),
            jax.ShapeDtypeStruct((T, 2), jnp.int32),
            jax.ShapeDtypeStruct((T, 2), jnp.float32),
            jax.ShapeDtypeStruct((NT, 1), jnp.int32),
            jax.ShapeDtypeStruct((NT, 1), jnp.int32),
        ],
    )(flat, gate_w)

    pos1 = pos2d[:, 0]
    pos2 = pos2d[:, 1]
    w1col = wn2d[:, 0:1]
    w2col = wn2d[:, 1:2]
    te = te2d.reshape(NT)
    act = act2d.reshape(NT)

    TPW = T // NW

    @functools.partial(
        pl.kernel,
        out_type=jax.ShapeDtypeStruct((P, D), jnp.float32),
        mesh=mesh,
        scratch_types=[
            pltpu.VMEM((TPW,), jnp.int32),
            pltpu.VMEM((TPW,), jnp.int32),
            pltpu.VMEM((TPW, D), jnp.float32),
            pltpu.SemaphoreType.DMA,
        ],
    )
    def _sc_dispatch(x_hbm, pos1_hbm, pos2_hbm, xs_hbm, idx1_v, idx2_v,
                     rows_v, sem):
        wid = lax.axis_index("s") * NC + lax.axis_index("c")
        base = wid * TPW
        pltpu.sync_copy(x_hbm.at[pl.ds(base, TPW)], rows_v)
        pltpu.sync_copy(pos1_hbm.at[pl.ds(base, TPW)], idx1_v)
        pltpu.sync_copy(pos2_hbm.at[pl.ds(base, TPW)], idx2_v)
        c1 = pltpu.async_copy(rows_v, xs_hbm.at[idx1_v], sem)
        c2 = pltpu.async_copy(rows_v, xs_hbm.at[idx2_v], sem)
        c1.wait()
        c2.wait()

    xs = _sc_dispatch(flat, pos1, pos2)

    grid_spec = pltpu.PrefetchScalarGridSpec(
        num_scalar_prefetch=2,
        grid=(NT,),
        in_specs=[
            pl.BlockSpec((BTS, D), lambda j, te_r, act_r: (j, 0)),
            pl.BlockSpec((1, I, D), lambda j, te_r, act_r: (te_r[j], 0, 0)),
            pl.BlockSpec((1, I, D), lambda j, te_r, act_r: (te_r[j], 0, 0)),
            pl.BlockSpec((1, D, I), lambda j, te_r, act_r: (te_r[j], 0, 0)),
        ],
        out_specs=pl.BlockSpec((BTS, D), lambda j, te_r, act_r: (j, 0)),
    )
    ysw = pl.pallas_call(
        _ffn_body,
        grid_spec=grid_spec,
        out_shape=jax.ShapeDtypeStruct((P, D), jnp.float32),
        compiler_params=pltpu.CompilerParams(
            dimension_semantics=("arbitrary",)),
    )(te, act, xs, expert_gate, expert_up, expert_down)

    TPW = T // NW
    CH2 = TPW // 2

    @functools.partial(
        pl.kernel,
        out_type=[
            jax.ShapeDtypeStruct((T, D), jnp.float32),
            jax.ShapeDtypeStruct((T, D), jnp.float32),
        ],
        mesh=mesh,
        scratch_types=[
            pltpu.VMEM((CH2,), jnp.int32),
            pltpu.VMEM((CH2, D), jnp.float32),
            pltpu.SemaphoreType.DMA,
        ],
    )
    def _sc_combine(ysw_hbm, pos1_hbm, pos2_hbm, moe1_hbm, moe2_hbm,
                    idx_v, rows_v, sem):
        wid = lax.axis_index("s") * NC + lax.axis_index("c")
        base = wid * TPW

        def body(c, carry):
            o = base + c * CH2
            pltpu.sync_copy(pos1_hbm.at[pl.ds(o, CH2)], idx_v)
            pltpu.async_copy(ysw_hbm.at[idx_v], rows_v, sem).wait()
            pltpu.sync_copy(rows_v, moe1_hbm.at[pl.ds(o, CH2)])
            pltpu.sync_copy(pos2_hbm.at[pl.ds(o, CH2)], idx_v)
            pltpu.async_copy(ysw_hbm.at[idx_v], rows_v, sem).wait()
            pltpu.sync_copy(rows_v, moe2_hbm.at[pl.ds(o, CH2)])
            return carry

        lax.fori_loop(0, TPW // CH2, body, 0)

    moe1, moe2 = _sc_combine(ysw, pos1, pos2)

    out = pl.pallas_call(
        _shared_body,
        grid=(T // BT,),
        in_specs=[
            pl.BlockSpec((BT, D), lambda j: (j, 0)),
            pl.BlockSpec((BT, D), lambda j: (j, 0)),
            pl.BlockSpec((BT, D), lambda j: (j, 0)),
            pl.BlockSpec((BT, 1), lambda j: (j, 0)),
            pl.BlockSpec((BT, 1), lambda j: (j, 0)),
            pl.BlockSpec((SH, D), lambda j: (0, 0)),
            pl.BlockSpec((SH, D), lambda j: (0, 0)),
            pl.BlockSpec((D, SH), lambda j: (0, 0)),
        ],
        out_specs=pl.BlockSpec((BT, D), lambda j: (j, 0)),
        out_shape=jax.ShapeDtypeStruct((T, D), jnp.float32),
        compiler_params=pltpu.CompilerParams(
            dimension_semantics=("parallel",)),
    )(flat, moe1, moe2, w1col, w2col, shared_gate_w, shared_up_w,
      shared_down_w)

    return out.reshape(bsz, seq, dim), logits


# speedup vs baseline: 1.8539x; 1.0057x over previous
import functools

import jax
import jax.numpy as jnp
from jax import lax
from jax.experimental import pallas as pl
from jax.experimental.pallas import tpu as pltpu
from jax.experimental.pallas import tpu_sc as plsc

_BT = 256
_BTS = 256


def _router_body(x_ref, gate_w_ref, logits_ref, pos1_ref, pos2_ref, w1_ref,
                 w2_ref, te_ref, act_ref):
    T = x_ref.shape[0]
    E = gate_w_ref.shape[0]
    BTS = _BTS
    NT = te_ref.shape[0]
    logits = jax.lax.dot_general(
        x_ref[...], gate_w_ref[...], (((1,), (1,)), ((), ())),
        preferred_element_type=jnp.float32)
    logits_ref[...] = logits
    s = jax.nn.sigmoid(logits)
    eidx = jax.lax.broadcasted_iota(jnp.int32, (T, E), 1)
    s_left = jnp.roll(s, -1, axis=1)
    s_right = jnp.roll(s, 1, axis=1)
    even = (eidx % 2) == 0
    gsc = s + jnp.where(even, s_left, s_right)
    glane = eidx >> 1
    m1 = jnp.max(gsc, axis=1, keepdims=True)
    i1g = jnp.min(jnp.where(gsc == m1, glane, 99), axis=1, keepdims=True)
    g2 = jnp.where(glane == i1g, -jnp.inf, gsc)
    m2 = jnp.max(g2, axis=1, keepdims=True)
    i2g = jnp.min(jnp.where(g2 == m2, glane, 99), axis=1, keepdims=True)
    emask = (glane == i1g) | (glane == i2g)
    ms = s * emask.astype(jnp.float32)
    v1 = jnp.max(ms, axis=1, keepdims=True)
    j1 = jnp.min(jnp.where(ms == v1, eidx, 99), axis=1, keepdims=True)
    ms2 = jnp.where(eidx == j1, -jnp.inf, ms)
    v2 = jnp.max(ms2, axis=1, keepdims=True)
    j2 = jnp.min(jnp.where(ms2 == v2, eidx, 99), axis=1, keepdims=True)
    denom = jnp.maximum(v1 + v2, 1e-12)
    w1_ref[...] = v1 / denom
    w2_ref[...] = v2 / denom

    H1 = (eidx == j1).astype(jnp.int32)
    H2 = (eidx == j2).astype(jnp.int32)
    Hsum = (H1 + H2).astype(jnp.float32)
    CB = 256
    ri = jax.lax.broadcasted_iota(jnp.int32, (CB, CB), 0)
    ci = jax.lax.broadcasted_iota(jnp.int32, (CB, CB), 1)
    tril = (ri > ci).astype(jnp.bfloat16)
    chunks = []
    carry = jnp.zeros((1, E), jnp.float32)
    for c in range(T // CB):
        Hc = Hsum[c * CB:(c + 1) * CB, :]
        Cc = jax.lax.dot_general(
            tril, Hc.astype(jnp.bfloat16), (((1,), (0,)), ((), ())),
            preferred_element_type=jnp.float32)
        chunks.append(Cc + carry)
        carry = carry + jnp.sum(Hc, axis=0, keepdims=True)
    C = jnp.concatenate(chunks, axis=0).astype(jnp.int32)
    cnt = carry.astype(jnp.int32)
    pc = ((cnt + (BTS - 1)) // BTS) * BTS
    lane8 = jax.lax.broadcasted_iota(jnp.int32, (1, E), 1)
    incl = pc
    for k in (1, 2, 4):
        sh = jnp.where(lane8 >= k, jnp.roll(incl, k, axis=1), 0)
        incl = incl + sh
    off = incl - pc
    rank1 = jnp.sum(H1 * C, axis=1, keepdims=True)
    rank2 = jnp.sum(H2 * C, axis=1, keepdims=True)
    pos1_ref[...] = jnp.sum(H1 * off, axis=1, keepdims=True) + rank1
    pos2_ref[...] = jnp.sum(H2 * off, axis=1, keepdims=True) + rank2

    jrow = jax.lax.broadcasted_iota(jnp.int32, (NT, E), 0) * BTS
    offb = jnp.broadcast_to(off, (NT, E))
    pcb = jnp.broadcast_to(pc, (NT, E))
    eb = jax.lax.broadcasted_iota(jnp.int32, (NT, E), 1)
    inseg = (jrow >= offb) & (jrow < offb + pcb)
    te_col = jnp.sum(jnp.where(inseg, eb, 0), axis=1, keepdims=True)
    total = jnp.sum(pc, axis=1, keepdims=True)
    te_last = jnp.max(jnp.where(pc > 0, lane8, 0), axis=1, keepdims=True)
    act_col = (jrow[:, 0:1] < jnp.broadcast_to(total, (NT, 1)))
    te_ref[...] = jnp.where(act_col, te_col, jnp.broadcast_to(te_last, (NT, 1)))
    act_ref[...] = act_col.astype(jnp.int32)


def _ffn_body(te_ref, act_ref, xs_ref, wg_ref, wu_ref, wd_ref,
              out_ref):
    j = pl.program_id(0)

    @pl.when(act_ref[j] == 1)
    def _():
        x = xs_ref[...]
        gate = jax.lax.dot_general(x, wg_ref[0], (((1,), (1,)), ((), ())),
                                   preferred_element_type=jnp.float32)
        up = jax.lax.dot_general(x, wu_ref[0], (((1,), (1,)), ((), ())),
                                 preferred_element_type=jnp.float32)
        h = gate * jax.nn.sigmoid(gate) * up
        y = jax.lax.dot_general(h, wd_ref[0], (((1,), (1,)), ((), ())),
                                preferred_element_type=jnp.float32)
        out_ref[...] = y


def _shared_body(x_ref, moe1_ref, moe2_ref, w1_ref, w2_ref, wg_ref, wu_ref,
                 wd_ref, out_ref):
    x = x_ref[...]
    gate = jax.lax.dot_general(x, wg_ref[...], (((1,), (1,)), ((), ())),
                               preferred_element_type=jnp.float32)
    up = jax.lax.dot_general(x, wu_ref[...], (((1,), (1,)), ((), ())),
                             preferred_element_type=jnp.float32)
    h = gate * jax.nn.sigmoid(gate) * up
    y = jax.lax.dot_general(h, wd_ref[...], (((1,), (1,)), ((), ())),
                            preferred_element_type=jnp.float32)
    out_ref[...] = (moe1_ref[...] * w1_ref[...] +
                    moe2_ref[...] * w2_ref[...]) + y


def kernel(hidden_states, gate_w, expert_gate, expert_up, expert_down,
           shared_gate_w, shared_up_w, shared_down_w):
    bsz, seq, dim = hidden_states.shape
    T = bsz * seq
    E, I, D = expert_gate.shape
    SH = shared_gate_w.shape[0]
    flat = hidden_states.reshape(T, D)
    BT, BTS = _BT, _BTS
    TK = 2 * T
    NT = TK // BTS + E
    P = NT * BTS

    NC, NS = 2, 16
    NW = NC * NS
    mesh = plsc.VectorSubcoreMesh(core_axis_name="c", subcore_axis_name="s",
                                  num_cores=NC, num_subcores=NS)

    logits, pos1o, pos2o, w1col, w2col, te2d, act2d = pl.pallas_call(
        _router_body,
        out_shape=[
            jax.ShapeDtypeStruct((T, E), jnp.float32),
            jax.ShapeDtypeStruct((T, 1), jnp.int32),
            jax.ShapeDtypeStruct((T, 1), jnp.int32),
            jax.ShapeDtypeStruct((T, 1), jnp.float32),
            jax.ShapeDtypeStruct((T, 1), jnp.float32),
            jax.ShapeDtypeStruct((NT, 1), jnp.int32),
            jax.ShapeDtypeStruct((NT, 1), jnp.int32),
        ],
    )(flat, gate_w)

    pos1 = pos1o.reshape(T)
    pos2 = pos2o.reshape(T)
    te = te2d.reshape(NT)
    act = act2d.reshape(NT)

    TPW = T // NW

    @functools.partial(
        pl.kernel,
        out_type=jax.ShapeDtypeStruct((P, D), jnp.float32),
        mesh=mesh,
        scratch_types=[
            pltpu.VMEM((TPW,), jnp.int32),
            pltpu.VMEM((TPW,), jnp.int32),
            pltpu.VMEM((TPW, D), jnp.float32),
            pltpu.SemaphoreType.DMA,
        ],
    )
    def _sc_dispatch(x_hbm, pos1_hbm, pos2_hbm, xs_hbm, idx1_v, idx2_v,
                     rows_v, sem):
        wid = lax.axis_index("s") * NC + lax.axis_index("c")
        base = wid * TPW
        pltpu.sync_copy(x_hbm.at[pl.ds(base, TPW)], rows_v)
        pltpu.sync_copy(pos1_hbm.at[pl.ds(base, TPW)], idx1_v)
        pltpu.sync_copy(pos2_hbm.at[pl.ds(base, TPW)], idx2_v)
        c1 = pltpu.async_copy(rows_v, xs_hbm.at[idx1_v], sem)
        c2 = pltpu.async_copy(rows_v, xs_hbm.at[idx2_v], sem)
        c1.wait()
        c2.wait()

    xs = _sc_dispatch(flat, pos1, pos2)

    grid_spec = pltpu.PrefetchScalarGridSpec(
        num_scalar_prefetch=2,
        grid=(NT,),
        in_specs=[
            pl.BlockSpec((BTS, D), lambda j, te_r, act_r: (j, 0)),
            pl.BlockSpec((1, I, D), lambda j, te_r, act_r: (te_r[j], 0, 0)),
            pl.BlockSpec((1, I, D), lambda j, te_r, act_r: (te_r[j], 0, 0)),
            pl.BlockSpec((1, D, I), lambda j, te_r, act_r: (te_r[j], 0, 0)),
        ],
        out_specs=pl.BlockSpec((BTS, D), lambda j, te_r, act_r: (j, 0)),
    )
    ysw = pl.pallas_call(
        _ffn_body,
        grid_spec=grid_spec,
        out_shape=jax.ShapeDtypeStruct((P, D), jnp.float32),
        compiler_params=pltpu.CompilerParams(
            dimension_semantics=("arbitrary",)),
    )(te, act, xs, expert_gate, expert_up, expert_down)

    TPW = T // NW
    CH2 = TPW // 2

    @functools.partial(
        pl.kernel,
        out_type=[
            jax.ShapeDtypeStruct((T, D), jnp.float32),
            jax.ShapeDtypeStruct((T, D), jnp.float32),
        ],
        mesh=mesh,
        scratch_types=[
            pltpu.VMEM((CH2,), jnp.int32),
            pltpu.VMEM((CH2, D), jnp.float32),
            pltpu.SemaphoreType.DMA,
        ],
    )
    def _sc_combine(ysw_hbm, pos1_hbm, pos2_hbm, moe1_hbm, moe2_hbm,
                    idx_v, rows_v, sem):
        wid = lax.axis_index("s") * NC + lax.axis_index("c")
        base = wid * TPW

        def body(c, carry):
            o = base + c * CH2
            pltpu.sync_copy(pos1_hbm.at[pl.ds(o, CH2)], idx_v)
            pltpu.async_copy(ysw_hbm.at[idx_v], rows_v, sem).wait()
            pltpu.sync_copy(rows_v, moe1_hbm.at[pl.ds(o, CH2)])
            pltpu.sync_copy(pos2_hbm.at[pl.ds(o, CH2)], idx_v)
            pltpu.async_copy(ysw_hbm.at[idx_v], rows_v, sem).wait()
            pltpu.sync_copy(rows_v, moe2_hbm.at[pl.ds(o, CH2)])
            return carry

        lax.fori_loop(0, TPW // CH2, body, 0)

    moe1, moe2 = _sc_combine(ysw, pos1, pos2)

    out = pl.pallas_call(
        _shared_body,
        grid=(T // BT,),
        in_specs=[
            pl.BlockSpec((BT, D), lambda j: (j, 0)),
            pl.BlockSpec((BT, D), lambda j: (j, 0)),
            pl.BlockSpec((BT, D), lambda j: (j, 0)),
            pl.BlockSpec((BT, 1), lambda j: (j, 0)),
            pl.BlockSpec((BT, 1), lambda j: (j, 0)),
            pl.BlockSpec((SH, D), lambda j: (0, 0)),
            pl.BlockSpec((SH, D), lambda j: (0, 0)),
            pl.BlockSpec((D, SH), lambda j: (0, 0)),
        ],
        out_specs=pl.BlockSpec((BT, D), lambda j: (j, 0)),
        out_shape=jax.ShapeDtypeStruct((T, D), jnp.float32),
        compiler_params=pltpu.CompilerParams(
            dimension_semantics=("parallel",)),
    )(flat, moe1, moe2, w1col, w2col, shared_gate_w, shared_up_w,
      shared_down_w)

    return out.reshape(bsz, seq, dim), logits
